# baseline (speedup 1.0000x reference)
import functools

import jax
import jax.numpy as jnp
from jax import lax
from jax.experimental import pallas as pl
from jax.experimental.pallas import tpu as pltpu
from jax.experimental.pallas import tpu_sc as plsc

B = 16384
D = 16
NC = 2
NS = 16
NW = NC * NS
BPW = B // NW
CH = 128
NCH = BPW // CH
LANES = 16


def _body(uidx_hbm, midx_hbm, ub_hbm, mb_hbm, ue_hbm, me_hbm, out_hbm,
          uidx, midx, ue_v, me_v, ub_v, mb_v, out_v, sem):
    wid = lax.axis_index("s") * NC + lax.axis_index("c")

    pltpu.sync_copy(uidx_hbm.at[pl.ds(wid * NCH, NCH)], uidx)
    pltpu.sync_copy(midx_hbm.at[pl.ds(wid * NCH, NCH)], midx)

    copies = []
    for j in range(NCH):
        sl = pl.ds(j * CH, CH)
        copies.append(pltpu.async_copy(ue_hbm.at[uidx.at[j]], ue_v.at[sl], sem))
        copies.append(pltpu.async_copy(me_hbm.at[midx.at[j]], me_v.at[sl], sem))
        copies.append(pltpu.async_copy(ub_hbm.at[uidx.at[j]], ub_v.at[sl], sem))
        copies.append(pltpu.async_copy(mb_hbm.at[midx.at[j]], mb_v.at[sl], sem))
    for c in copies:
        c.wait()

    lanes = lax.iota(jnp.int32, LANES)

    def permute(v, idx):
        return lax.gather(
            v, idx[:, None],
            lax.GatherDimensionNumbers(
                offset_dims=(), collapsed_slice_dims=(0,),
                start_index_map=(0,)),
            (1,), mode=lax.GatherScatterMode.PROMISE_IN_BOUNDS)

    def blk_body(blk, _):
        sl16 = pl.ds(blk * LANES, LANES)
        acc = jnp.zeros((LANES,), jnp.float32)
        for i in range(LANES):
            r = blk * LANES + i
            p = ue_v[r, :] * me_v[r, :]
            for s in (1, 2, 4, 8):
                p = p + permute(p, lanes ^ s)
            acc = jnp.where(lanes == i, p, acc)
        out_v[sl16] = acc + ub_v[sl16] + mb_v[sl16]
        return _

    lax.fori_loop(0, BPW // LANES, blk_body, 0)

    pltpu.sync_copy(out_v, out_hbm.at[pl.ds(wid * BPW, BPW)])


@jax.jit
def _run(uidx, midx, ub, mb, ue, me):
    mesh = plsc.VectorSubcoreMesh(
        core_axis_name="c", subcore_axis_name="s",
        num_cores=NC, num_subcores=NS)
    f = pl.kernel(
        _body,
        out_type=jax.ShapeDtypeStruct((B,), jnp.float32),
        mesh=mesh,
        scratch_types=[
            pltpu.VMEM((NCH, CH), jnp.int32),
            pltpu.VMEM((NCH, CH), jnp.int32),
            pltpu.VMEM((BPW, D), jnp.float32),
            pltpu.VMEM((BPW, D), jnp.float32),
            pltpu.VMEM((BPW,), jnp.float32),
            pltpu.VMEM((BPW,), jnp.float32),
            pltpu.VMEM((BPW,), jnp.float32),
            pltpu.SemaphoreType.DMA,
        ],
        compiler_params=pltpu.CompilerParams(use_tc_tiling_on_sc=False),
    )
    return f(uidx, midx, ub, mb, ue, me)


def kernel(user_id, movie_id, user_bias_table, movie_bias_table,
           user_emb_table, movie_emb_table):
    uidx = user_id.astype(jnp.int32).reshape(NW * NCH, CH)
    midx = movie_id.astype(jnp.int32).reshape(NW * NCH, CH)
    out = _run(uidx, midx, user_bias_table.reshape(-1),
               movie_bias_table.reshape(-1),
               user_emb_table, movie_emb_table)
    return out.reshape(B, 1)
